# baseline (speedup 1.0000x reference)
import jax
import jax.numpy as jnp
import numpy as np
from jax.experimental import pallas as pl
from jax.experimental.pallas import tpu as pltpu

R = 512
G = R // 16
N1 = 16384
STEPS = N1 // R

_S_SEL = jnp.asarray(np.repeat(np.eye(G, dtype=np.float32), 16, axis=1) / 16.0)


def _lrelu(x):
    return jnp.where(x > 0, x, 0.01 * x)


def _mean16_lanes(xw):
    parts = [xw[:, k * 256:(k + 1) * 256] for k in range(16)]
    while len(parts) > 1:
        parts = [parts[i] + parts[i + 1] for i in range(0, len(parts), 2)]
    return parts[0] * (1.0 / 16.0)


def _sage_kernel(x2_ref, x1_ref, x0_ref, S_ref, Wn0_ref, Ws0_ref, Wn1_ref,
                 Ws1_ref, out_ref, b_acc, a0_acc):
    i = pl.program_id(0)
    Wn0 = Wn0_ref[...]
    Ws0 = Ws0_ref[...]
    S = S_ref[...]

    a1 = _mean16_lanes(x2_ref[...])
    x1b = x1_ref[...]
    h1 = _lrelu(
        jnp.dot(x1b, Ws0, preferred_element_type=jnp.float32)
        + jnp.dot(a1, Wn0, preferred_element_type=jnp.float32))
    b_acc[pl.ds(i * G, G), :] = jnp.dot(S, h1,
                                        preferred_element_type=jnp.float32)
    a0_acc[pl.ds(i * G, G), :] = jnp.dot(S, x1b,
                                         preferred_element_type=jnp.float32)

    @pl.when(i == STEPS - 1)
    def _final():
        x0 = x0_ref[...]
        h0 = _lrelu(
            jnp.dot(x0, Ws0, preferred_element_type=jnp.float32)
            + jnp.dot(a0_acc[...], Wn0, preferred_element_type=jnp.float32))
        out_ref[...] = (
            jnp.dot(h0, Ws1_ref[...], preferred_element_type=jnp.float32)
            + jnp.dot(b_acc[...], Wn1_ref[...],
                      preferred_element_type=jnp.float32))


def kernel(x0, x1, x2, Wn0, Ws0, Wn1, Ws1):
    x2w = x2.reshape(N1, 16 * 256)
    return pl.pallas_call(
        _sage_kernel,
        grid=(STEPS,),
        in_specs=[
            pl.BlockSpec((R, 16 * 256), lambda i: (i, 0)),
            pl.BlockSpec((R, 256), lambda i: (i, 0)),
            pl.BlockSpec((1024, 256), lambda i: (0, 0)),
            pl.BlockSpec((G, R), lambda i: (0, 0)),
            pl.BlockSpec((256, 256), lambda i: (0, 0)),
            pl.BlockSpec((256, 256), lambda i: (0, 0)),
            pl.BlockSpec((256, 128), lambda i: (0, 0)),
            pl.BlockSpec((256, 128), lambda i: (0, 0)),
        ],
        out_specs=pl.BlockSpec((1024, 128), lambda i: (0, 0)),
        out_shape=jax.ShapeDtypeStruct((1024, 128), jnp.float32),
        scratch_shapes=[
            pltpu.VMEM((1024, 256), jnp.float32),
            pltpu.VMEM((1024, 256), jnp.float32),
        ],
    )(x2w, x1, x0, _S_SEL, Wn0, Ws0, Wn1, Ws1)


# speedup vs baseline: 4.0967x; 4.0967x over previous
import jax
import jax.numpy as jnp
import numpy as np
from jax.experimental import pallas as pl
from jax.experimental.pallas import tpu as pltpu

R = 512
G = R // 16
N1 = 16384
STEPS = N1 // R

_S_SEL = jnp.asarray(np.repeat(np.eye(G, dtype=np.float32), 16, axis=1) / 16.0)


def _lrelu(x):
    return jnp.where(x > 0, x, 0.01 * x)


def _sage_kernel(x2_ref, x1_ref, x0_ref, S_ref, Wn0_ref, Ws0_ref, Wn1_ref,
                 Ws1_ref, out_ref, b_acc, a0_acc):
    i = pl.program_id(0)
    Wn0 = Wn0_ref[...]
    Ws0 = Ws0_ref[...]
    S = S_ref[...]

    x2b = x2_ref[...]
    a1 = jnp.mean(x2b.reshape(R, 16, 256), axis=1)
    x1b = x1_ref[...]
    h1 = _lrelu(
        jnp.dot(x1b, Ws0, preferred_element_type=jnp.float32)
        + jnp.dot(a1, Wn0, preferred_element_type=jnp.float32))
    b_acc[pl.ds(i * G, G), :] = jnp.dot(S, h1,
                                        preferred_element_type=jnp.float32)
    a0_acc[pl.ds(i * G, G), :] = jnp.dot(S, x1b,
                                         preferred_element_type=jnp.float32)

    @pl.when(i == STEPS - 1)
    def _final():
        x0 = x0_ref[...]
        h0 = _lrelu(
            jnp.dot(x0, Ws0, preferred_element_type=jnp.float32)
            + jnp.dot(a0_acc[...], Wn0, preferred_element_type=jnp.float32))
        out_ref[...] = (
            jnp.dot(h0, Ws1_ref[...], preferred_element_type=jnp.float32)
            + jnp.dot(b_acc[...], Wn1_ref[...],
                      preferred_element_type=jnp.float32))


def kernel(x0, x1, x2, Wn0, Ws0, Wn1, Ws1):
    return pl.pallas_call(
        _sage_kernel,
        grid=(STEPS,),
        in_specs=[
            pl.BlockSpec((R * 16, 256), lambda i: (i, 0)),
            pl.BlockSpec((R, 256), lambda i: (i, 0)),
            pl.BlockSpec((1024, 256), lambda i: (0, 0)),
            pl.BlockSpec((G, R), lambda i: (0, 0)),
            pl.BlockSpec((256, 256), lambda i: (0, 0)),
            pl.BlockSpec((256, 256), lambda i: (0, 0)),
            pl.BlockSpec((256, 128), lambda i: (0, 0)),
            pl.BlockSpec((256, 128), lambda i: (0, 0)),
        ],
        out_specs=pl.BlockSpec((1024, 128), lambda i: (0, 0)),
        out_shape=jax.ShapeDtypeStruct((1024, 128), jnp.float32),
        scratch_shapes=[
            pltpu.VMEM((1024, 256), jnp.float32),
            pltpu.VMEM((1024, 256), jnp.float32),
        ],
    )(x2, x1, x0, _S_SEL, Wn0, Ws0, Wn1, Ws1)


# speedup vs baseline: 4.1063x; 1.0023x over previous
import jax
import jax.numpy as jnp
import numpy as np
from jax.experimental import pallas as pl
from jax.experimental.pallas import tpu as pltpu

R = 1024
G = R // 16
N1 = 16384
STEPS = N1 // R

_S_SEL = jnp.asarray(np.repeat(np.eye(G, dtype=np.float32), 16, axis=1) / 16.0)


def _lrelu(x):
    return jnp.where(x > 0, x, 0.01 * x)


def _sage_kernel(x2_ref, x1_ref, x0_ref, S_ref, Wn0_ref, Ws0_ref, Wn1_ref,
                 Ws1_ref, out_ref, b_acc, a0_acc):
    i = pl.program_id(0)
    Wn0 = Wn0_ref[...]
    Ws0 = Ws0_ref[...]
    S = S_ref[...]

    x2b = x2_ref[...]
    a1 = jnp.mean(x2b.reshape(R, 16, 256), axis=1)
    x1b = x1_ref[...]
    h1 = _lrelu(
        jnp.dot(x1b, Ws0, preferred_element_type=jnp.float32)
        + jnp.dot(a1, Wn0, preferred_element_type=jnp.float32))
    b_acc[pl.ds(i * G, G), :] = jnp.dot(S, h1,
                                        preferred_element_type=jnp.float32)
    a0_acc[pl.ds(i * G, G), :] = jnp.dot(S, x1b,
                                         preferred_element_type=jnp.float32)

    @pl.when(i == STEPS - 1)
    def _final():
        x0 = x0_ref[...]
        h0 = _lrelu(
            jnp.dot(x0, Ws0, preferred_element_type=jnp.float32)
            + jnp.dot(a0_acc[...], Wn0, preferred_element_type=jnp.float32))
        out_ref[...] = (
            jnp.dot(h0, Ws1_ref[...], preferred_element_type=jnp.float32)
            + jnp.dot(b_acc[...], Wn1_ref[...],
                      preferred_element_type=jnp.float32))


def kernel(x0, x1, x2, Wn0, Ws0, Wn1, Ws1):
    return pl.pallas_call(
        _sage_kernel,
        grid=(STEPS,),
        in_specs=[
            pl.BlockSpec((R * 16, 256), lambda i: (i, 0)),
            pl.BlockSpec((R, 256), lambda i: (i, 0)),
            pl.BlockSpec((1024, 256), lambda i: (0, 0)),
            pl.BlockSpec((G, R), lambda i: (0, 0)),
            pl.BlockSpec((256, 256), lambda i: (0, 0)),
            pl.BlockSpec((256, 256), lambda i: (0, 0)),
            pl.BlockSpec((256, 128), lambda i: (0, 0)),
            pl.BlockSpec((256, 128), lambda i: (0, 0)),
        ],
        out_specs=pl.BlockSpec((1024, 128), lambda i: (0, 0)),
        out_shape=jax.ShapeDtypeStruct((1024, 128), jnp.float32),
        scratch_shapes=[
            pltpu.VMEM((1024, 256), jnp.float32),
            pltpu.VMEM((1024, 256), jnp.float32),
        ],
    )(x2, x1, x0, _S_SEL, Wn0, Ws0, Wn1, Ws1)
